# Initial kernel scaffold; baseline (speedup 1.0000x reference)
import jax
import jax.numpy as jnp
from jax.experimental import pallas as pl


def kernel(x, edge_index, edge_attr, W1, att_src1, att_dst1, We1, att_edge1, b1, W2, att_src2, att_dst2, We2, att_edge2, b2):
    raise NotImplementedError("write your pallas kernel here")


# speedup vs baseline: 18.2697x; 18.2697x over previous
import functools

import jax
import jax.numpy as jnp
from jax import lax
from jax.experimental import pallas as pl
from jax.experimental.pallas import tpu as pltpu
from jax.experimental.pallas import tpu_sc as plsc

N = 10000
E = 160000
D = 64
H = 16
C = 64
HC = H * C

NC = 2
NS = 16
NW = NC * NS

R = 40
ROWS = E // R
P1_ROWS = ROWS // NS
P2_ROWS = ROWS // NW
NT = N // NS

_f32 = jnp.float32


def _dense_node_body(x_ref, w_ref, asv_ref, adv_ref, xh_ref, as_ref, ad_ref):
    xb = x_ref[...]
    xh = jnp.dot(xb, w_ref[...], preferred_element_type=_f32)
    xh_ref[...] = xh
    xh3 = xh.reshape(xb.shape[0], H, C)
    as_ref[...] = jnp.sum(xh3 * asv_ref[...][None], axis=-1)
    ad_ref[...] = jnp.sum(xh3 * adv_ref[...][None], axis=-1)


def _dense_node(x, w, att_src, att_dst):
    n = x.shape[0]
    blk = 1000
    grid = n // blk
    return pl.pallas_call(
        _dense_node_body,
        grid=(grid,),
        in_specs=[
            pl.BlockSpec((blk, D), lambda i: (i, 0)),
            pl.BlockSpec((D, HC), lambda i: (0, 0)),
            pl.BlockSpec((H, C), lambda i: (0, 0)),
            pl.BlockSpec((H, C), lambda i: (0, 0)),
        ],
        out_specs=[
            pl.BlockSpec((blk, HC), lambda i: (i, 0)),
            pl.BlockSpec((blk, H), lambda i: (i, 0)),
            pl.BlockSpec((blk, H), lambda i: (i, 0)),
        ],
        out_shape=[
            jax.ShapeDtypeStruct((n, HC), _f32),
            jax.ShapeDtypeStruct((n, H), _f32),
            jax.ShapeDtypeStruct((n, H), _f32),
        ],
    )(x, w, att_src, att_dst)


def _combine_dense_body(p_ref, b_ref, w_ref, asv_ref, adv_ref,
                        xh_ref, as_ref, ad_ref):
    h1 = jnp.maximum(p_ref[0] + p_ref[1] + b_ref[...][None, :], 0.0)
    xh = jnp.dot(h1, w_ref[...], preferred_element_type=_f32)
    xh_ref[...] = xh
    xh3 = xh.reshape(h1.shape[0], H, C)
    as_ref[...] = jnp.sum(xh3 * asv_ref[...][None], axis=-1)
    ad_ref[...] = jnp.sum(xh3 * adv_ref[...][None], axis=-1)


def _combine_dense(parts, b, w, att_src, att_dst):
    blk = 1000
    grid = N // blk
    return pl.pallas_call(
        _combine_dense_body,
        grid=(grid,),
        in_specs=[
            pl.BlockSpec((2, blk, C), lambda i: (0, i, 0)),
            pl.BlockSpec((C,), lambda i: (0,)),
            pl.BlockSpec((C, HC), lambda i: (0, 0)),
            pl.BlockSpec((H, C), lambda i: (0, 0)),
            pl.BlockSpec((H, C), lambda i: (0, 0)),
        ],
        out_specs=[
            pl.BlockSpec((blk, HC), lambda i: (i, 0)),
            pl.BlockSpec((blk, H), lambda i: (i, 0)),
            pl.BlockSpec((blk, H), lambda i: (i, 0)),
        ],
        out_shape=[
            jax.ShapeDtypeStruct((N, HC), _f32),
            jax.ShapeDtypeStruct((N, H), _f32),
            jax.ShapeDtypeStruct((N, H), _f32),
        ],
    )(parts, b, w, att_src, att_dst)


def _edge_dense_body(ea_ref, w1_ref, w2_ref, o1_ref, o2_ref):
    eb = ea_ref[...]
    o1_ref[...] = jnp.dot(eb, w1_ref[...], preferred_element_type=_f32)
    o2_ref[...] = jnp.dot(eb, w2_ref[...], preferred_element_type=_f32)


def _edge_dense(edge_attr, wev1, wev2):
    blk = 8000
    grid = E // blk
    return pl.pallas_call(
        _edge_dense_body,
        grid=(grid,),
        in_specs=[
            pl.BlockSpec((blk, D), lambda i: (i, 0)),
            pl.BlockSpec((D, H), lambda i: (0, 0)),
            pl.BlockSpec((D, H), lambda i: (0, 0)),
        ],
        out_specs=[
            pl.BlockSpec((blk, H), lambda i: (i, 0)),
            pl.BlockSpec((blk, H), lambda i: (i, 0)),
        ],
        out_shape=[
            jax.ShapeDtypeStruct((E, H), _f32),
            jax.ShapeDtypeStruct((E, H), _f32),
        ],
    )(edge_attr, wev1, wev2)


def _combine_body(p_ref, b_ref, o_ref):
    o_ref[...] = jnp.maximum(p_ref[0] + p_ref[1] + b_ref[...][None, :], 0.0)


def _combine(parts, b):
    blk = 1000
    grid = N // blk
    return pl.pallas_call(
        _combine_body,
        grid=(grid,),
        in_specs=[
            pl.BlockSpec((2, blk, C), lambda i: (0, i, 0)),
            pl.BlockSpec((C,), lambda i: (0,)),
        ],
        out_specs=pl.BlockSpec((blk, C), lambda i: (i, 0)),
        out_shape=jax.ShapeDtypeStruct((N, C), _f32),
    )(parts, b)


def _zero16():
    return jnp.zeros((16,), _f32)


def _gat_sc_kernel(src_ref, dst_ref, asrc_ref, adst_ref, ae_ref, xh_ref,
                   outp_ref, ex_ref, inv0_ref, inv1_ref,
                   idxs_v, idxd_v, rows_s, rows_d, rows_e, exb,
                   xh_v, ex2, inv2, coefb, outb, zbuf, nbuf,
                   den_sp, out_sp,
                   semA, semB, semC, semX, semE, semI):
    cid = lax.axis_index("c")
    sid = lax.axis_index("s")
    wid = cid * NS + sid

    for i in range(125):
        for k in range(4):
            zbuf[i, pl.ds(16 * k, 16)] = _zero16()
    for i in range(125):
        nbuf[i, :] = _zero16()

    def _zero_body(i, _):
        r0 = sid * NT + i * 125
        pltpu.sync_copy(zbuf, out_sp.at[pl.ds(r0, 125)])
        pltpu.sync_copy(nbuf, den_sp.at[pl.ds(r0, 125)])
        return _

    lax.fori_loop(0, NT // 125, _zero_body, None)
    plsc.subcore_barrier()

    pltpu.sync_copy(src_ref.at[pl.ds(sid * P1_ROWS, P1_ROWS)],
                    idxs_v.at[pl.ds(0, P1_ROWS)])
    pltpu.sync_copy(dst_ref.at[pl.ds(sid * P1_ROWS, P1_ROWS)],
                    idxd_v.at[pl.ds(0, P1_ROWS)])

    def _p1_body(j, _):
        row = sid * P1_ROWS + j
        cA = pltpu.async_copy(asrc_ref.at[idxs_v.at[j]], rows_s, semA)
        cB = pltpu.async_copy(adst_ref.at[idxd_v.at[j]], rows_d, semB)
        cC = pltpu.async_copy(ae_ref.at[pl.ds(row * R, R)], rows_e, semC)
        cA.wait()
        cB.wait()
        cC.wait()
        for i in range(R):
            v = rows_s[i, :] + rows_d[i, :] + rows_e[i, :]
            v = jnp.where(v >= 0.0, v, 0.2 * v)
            exb[i, :] = jnp.exp(v)
        pltpu.sync_copy(exb, ex_ref.at[pl.ds(row * R, R)])
        pltpu.sync_copy(exb, den_sp.at[idxd_v.at[j]], add=True)
        return _

    lax.fori_loop(0, P1_ROWS, _p1_body, None)
    plsc.subcore_barrier()

    def _inv_body(i, _):
        r0 = sid * NT + i * 125
        pltpu.sync_copy(den_sp.at[pl.ds(r0, 125)], nbuf)
        for r in range(125):
            nbuf[r, :] = 0.0625 / (nbuf[r, :] + 1e-16)

        @pl.when(cid == 0)
        def _():
            pltpu.sync_copy(nbuf, inv0_ref.at[pl.ds(r0, 125)])

        @pl.when(cid == 1)
        def _():
            pltpu.sync_copy(nbuf, inv1_ref.at[pl.ds(r0, 125)])

        return _

    lax.fori_loop(0, NT // 125, _inv_body, None)
    plsc.subcore_barrier()

    pltpu.sync_copy(src_ref.at[pl.ds(wid * P2_ROWS, P2_ROWS)],
                    idxs_v.at[pl.ds(0, P2_ROWS)])
    pltpu.sync_copy(dst_ref.at[pl.ds(wid * P2_ROWS, P2_ROWS)],
                    idxd_v.at[pl.ds(0, P2_ROWS)])

    def _p2_body(j, _):
        row = wid * P2_ROWS + j
        cX = pltpu.async_copy(xh_ref.at[idxs_v.at[j]], xh_v, semX)
        cE = pltpu.async_copy(ex_ref.at[pl.ds(row * R, R)], ex2, semE)

        @pl.when(cid == 0)
        def _():
            pltpu.async_copy(inv0_ref.at[idxd_v.at[j]], inv2, semI)

        @pl.when(cid == 1)
        def _():
            pltpu.async_copy(inv1_ref.at[idxd_v.at[j]], inv2, semI)

        cE.wait()
        pltpu.make_async_copy(inv0_ref.at[idxd_v.at[j]], inv2, semI).wait()
        for i in range(R):
            coefb[i, :] = ex2[i, :] * inv2[i, :]
        cX.wait()

        def _edge_body(e, _):
            cf = coefb[e, :]
            for k in range(4):
                acc = _zero16()
                for h in range(H):
                    acc = acc + cf[h] * xh_v[e, pl.ds(h * C + 16 * k, 16)]
                outb[e, pl.ds(16 * k, 16)] = acc
            return _

        lax.fori_loop(0, R, _edge_body, None)
        pltpu.sync_copy(outb, out_sp.at[idxd_v.at[j]], add=True)
        return _

    lax.fori_loop(0, P2_ROWS, _p2_body, None)
    plsc.subcore_barrier()

    def _out_body(i, _):
        r0 = sid * NT + i * 125
        pltpu.sync_copy(out_sp.at[pl.ds(r0, 125)], zbuf)
        pltpu.sync_copy(zbuf, outp_ref.at[pl.ds(cid * N + r0, 125)])
        return _

    lax.fori_loop(0, NT // 125, _out_body, None)


def _gat_sc(srcA, dstA, a_src, a_dst, a_e, xh):
    mesh = plsc.VectorSubcoreMesh(core_axis_name="c", subcore_axis_name="s",
                                  num_cores=NC, num_subcores=NS)
    f = functools.partial(
        pl.kernel,
        out_type=[
            jax.ShapeDtypeStruct((2 * N, C), _f32),
            jax.ShapeDtypeStruct((E, H), _f32),
            jax.ShapeDtypeStruct((N, H), _f32),
            jax.ShapeDtypeStruct((N, H), _f32),
        ],
        mesh=mesh,
        compiler_params=pltpu.CompilerParams(use_tc_tiling_on_sc=False),
        scratch_types=[
            pltpu.VMEM((P1_ROWS, R), jnp.int32),
            pltpu.VMEM((P1_ROWS, R), jnp.int32),
            pltpu.VMEM((R, H), _f32),
            pltpu.VMEM((R, H), _f32),
            pltpu.VMEM((R, H), _f32),
            pltpu.VMEM((R, H), _f32),
            pltpu.VMEM((R, HC), _f32),
            pltpu.VMEM((R, H), _f32),
            pltpu.VMEM((R, H), _f32),
            pltpu.VMEM((R, H), _f32),
            pltpu.VMEM((R, C), _f32),
            pltpu.VMEM((125, C), _f32),
            pltpu.VMEM((125, H), _f32),
            pltpu.VMEM_SHARED((N, H), _f32),
            pltpu.VMEM_SHARED((N, C), _f32),
            pltpu.SemaphoreType.DMA,
            pltpu.SemaphoreType.DMA,
            pltpu.SemaphoreType.DMA,
            pltpu.SemaphoreType.DMA,
            pltpu.SemaphoreType.DMA,
            pltpu.SemaphoreType.DMA,
        ],
    )(_gat_sc_kernel)
    return f(srcA, dstA, a_src, a_dst, a_e, xh)


def kernel(x, edge_index, edge_attr,
           W1, att_src1, att_dst1, We1, att_edge1, b1,
           W2, att_src2, att_dst2, We2, att_edge2, b2):
    src = edge_index[0].astype(jnp.int32)
    dst = edge_index[1].astype(jnp.int32)
    srcA = src.reshape(ROWS, R)
    dstA = dst.reshape(ROWS, R)

    wev1 = jnp.sum(We1.reshape(D, H, C) * att_edge1[None], axis=-1)
    wev2 = jnp.sum(We2.reshape(D, H, C) * att_edge2[None], axis=-1)

    ae1, ae2 = _edge_dense(edge_attr, wev1, wev2)
    xh1, as1, ad1 = _dense_node(x, W1, att_src1, att_dst1)
    p1, _, _, _ = _gat_sc(srcA, dstA, as1, ad1, ae1, xh1)
    xh2, as2, ad2 = _combine_dense(p1.reshape(2, N, C), b1, W2,
                                   att_src2, att_dst2)
    p2, _, _, _ = _gat_sc(srcA, dstA, as2, ad2, ae2, xh2)
    return _combine(p2.reshape(2, N, C), b2)
